# baseline (speedup 1.0000x reference)
import functools

import jax
import jax.numpy as jnp
from jax import lax
from jax.experimental import pallas as pl
from jax.experimental.pallas import tpu as pltpu
from jax.experimental.pallas import tpu_sc as plsc

N = 10000
E = 320000
D = 128

NC = 2
NS = 16
K = 128
C = 80
IB = 8
NBUF = 2
E_PAD = NC * NS * C * K
N_ACC = 10240
ZR = N_ACC // NS

_MESH = plsc.VectorSubcoreMesh(core_axis_name="c", subcore_axis_name="s")


@functools.partial(
    pl.kernel,
    out_type=jax.ShapeDtypeStruct((NC, N_ACC, D), jnp.float32),
    mesh=_MESH,
    scratch_types=[
        pltpu.VMEM((2, IB, K), jnp.int32),
        pltpu.VMEM((2, IB, K), jnp.int32),
        pltpu.VMEM((K, D), jnp.float32),
        pltpu.VMEM((K, D), jnp.float32),
        pltpu.VMEM_SHARED((N_ACC, D), jnp.float32),
        [pltpu.SemaphoreType.DMA] * NBUF,
        [pltpu.SemaphoreType.DMA] * NBUF,
        pltpu.SemaphoreType.DMA,
        pltpu.SemaphoreType.DMA,
    ],
)
def _sc_aggregate(s_hbm, src_hbm, dst_hbm, out_hbm,
                  src_v, dst_v, buf0, buf1, acc,
                  gsem, ssem, isem1, isem2):
    c = lax.axis_index("c")
    s = lax.axis_index("s")
    wid = c * NS + s
    bufs = [buf0, buf1]

    zero = jnp.zeros((16,), jnp.float32)

    def _zbuf(i, _):
        buf0[i // (D // 16), pl.ds((i % (D // 16)) * 16, 16)] = zero
        return 0
    lax.fori_loop(0, K * (D // 16), _zbuf, 0)

    for r in range(ZR // K):
        pltpu.sync_copy(buf0, acc.at[pl.ds(s * ZR + r * K, K)])
    assert ZR % K == 0
    plsc.subcore_barrier()

    def _round(p_sc, p_g, scat_base, gath_base):
        scats = []
        for b in range(NBUF):
            pltpu.make_async_copy(s_hbm.at[pl.ds(0, K)], bufs[b],
                                  gsem[b]).wait()
            scats.append(pltpu.async_copy(
                bufs[b], acc.at[dst_v.at[p_sc].at[scat_base + b]], ssem[b],
                add=True))
        for b in range(NBUF):
            scats[b].wait()
            pltpu.async_copy(s_hbm.at[src_v.at[p_g].at[gath_base + b]],
                             bufs[b], gsem[b])

    def _blockbody(p, bidx):
        base_next = wid * C + (bidx + 1) * IB
        r1 = pltpu.async_copy(src_hbm.at[pl.ds(base_next, IB)],
                              src_v.at[1 - p], isem1)
        r2 = pltpu.async_copy(dst_hbm.at[pl.ds(base_next, IB)],
                              dst_v.at[1 - p], isem2)

        def _q(q, _):
            _round(p, p, NBUF * q, NBUF * q + NBUF)
            return 0
        lax.fori_loop(0, IB // NBUF - 1, _q, 0)
        r1.wait()
        r2.wait()
        _round(p, 1 - p, IB - NBUF, 0)

    pltpu.sync_copy(src_hbm.at[pl.ds(wid * C, IB)], src_v.at[0])
    pltpu.sync_copy(dst_hbm.at[pl.ds(wid * C, IB)], dst_v.at[0])
    for b in range(NBUF):
        pltpu.async_copy(s_hbm.at[src_v.at[0].at[b]], bufs[b], gsem[b])

    def _pair(v, _):
        _blockbody(0, 2 * v)
        _blockbody(1, 2 * v + 1)
        return 0
    lax.fori_loop(0, C // IB // 2, _pair, 0)

    for b in range(NBUF):
        pltpu.make_async_copy(s_hbm.at[pl.ds(0, K)], bufs[b], gsem[b]).wait()

    plsc.subcore_barrier()

    pltpu.sync_copy(acc.at[pl.ds(s * ZR, ZR)], out_hbm.at[c].at[pl.ds(s * ZR, ZR)])


def _tc12_body(agg_ref, w1_ref, b1_ref, w2_ref, out_ref):
    a = agg_ref[0] + agg_ref[1]
    h = jnp.maximum(
        jnp.dot(a, w1_ref[...], preferred_element_type=jnp.float32,
                precision=lax.Precision.HIGHEST) + b1_ref[...], 0.0)
    out_ref[...] = jnp.dot(h, w2_ref[...], preferred_element_type=jnp.float32,
                           precision=lax.Precision.HIGHEST)


def _tc3_body(agg_ref, b2_ref, w3_ref, out_ref):
    h = jnp.maximum(agg_ref[0] + agg_ref[1] + b2_ref[...], 0.0)
    out_ref[...] = jnp.dot(h, w3_ref[...], preferred_element_type=jnp.float32,
                           precision=lax.Precision.HIGHEST)


def _tc_final_body(agg_ref, b3_ref, out_ref):
    out_ref[...] = agg_ref[0] + agg_ref[1] + b3_ref[...]


_R = 1000


def _tc12(aggx, w1, b1, w2):
    h1w = w1.shape[1]
    return pl.pallas_call(
        _tc12_body,
        grid=(N // _R,),
        in_specs=[
            pl.BlockSpec((NC, _R, D), lambda i: (0, i, 0)),
            pl.BlockSpec((D, h1w), lambda i: (0, 0)),
            pl.BlockSpec((1, h1w), lambda i: (0, 0)),
            pl.BlockSpec((h1w, D), lambda i: (0, 0)),
        ],
        out_specs=pl.BlockSpec((_R, D), lambda i: (i, 0)),
        out_shape=jax.ShapeDtypeStruct((N, D), jnp.float32),
    )(aggx, w1, b1.reshape(1, h1w), w2)


def _tc3(agg2, b2, w3):
    return pl.pallas_call(
        _tc3_body,
        grid=(N // _R,),
        in_specs=[
            pl.BlockSpec((NC, _R, D), lambda i: (0, i, 0)),
            pl.BlockSpec((1, D), lambda i: (0, 0)),
            pl.BlockSpec((D, D), lambda i: (0, 0)),
        ],
        out_specs=pl.BlockSpec((_R, D), lambda i: (i, 0)),
        out_shape=jax.ShapeDtypeStruct((N, D), jnp.float32),
    )(agg2, b2.reshape(1, D), w3)


def _tc_final(agg3, b3):
    return pl.pallas_call(
        _tc_final_body,
        grid=(N // _R,),
        in_specs=[
            pl.BlockSpec((NC, _R, D), lambda i: (0, i, 0)),
            pl.BlockSpec((1, D), lambda i: (0, 0)),
        ],
        out_specs=pl.BlockSpec((_R, D), lambda i: (i, 0)),
        out_shape=jax.ShapeDtypeStruct((N, D), jnp.float32),
    )(agg3, b3.reshape(1, D))


def kernel(x, adj, W1, b1, W2, b2, W3, b3):
    src = adj[0]
    dst = adj[1]
    nw = NC * NS
    per = E // nw
    pad_per = C * K - per
    src_p = jnp.pad(src.reshape(nw, per), ((0, 0), (0, pad_per))).reshape(-1, K)
    dummy = jnp.broadcast_to(
        (N + 16 + jnp.arange(nw, dtype=jnp.int32))[:, None], (nw, pad_per))
    dst_p = jnp.concatenate(
        [dst.reshape(nw, per), dummy], axis=1).reshape(-1, K)
    src_p = jnp.pad(src_p, ((0, IB), (0, 0)))
    dst_p = jnp.pad(dst_p, ((0, IB), (0, 0)))

    aggx = _sc_aggregate(x, src_p, dst_p)
    s2 = _tc12(aggx, W1, b1, W2)
    agg2 = _sc_aggregate(s2, src_p, dst_p)
    s3 = _tc3(agg2, b2, W3)
    agg3 = _sc_aggregate(s3, src_p, dst_p)
    return _tc_final(agg3, b3)


# speedup vs baseline: 1.0633x; 1.0633x over previous
import functools

import jax
import jax.numpy as jnp
from jax import lax
from jax.experimental import pallas as pl
from jax.experimental.pallas import tpu as pltpu
from jax.experimental.pallas import tpu_sc as plsc

N = 10000
E = 320000
D = 128

NC = 2
NS = 16
K = 64
C = 160
IB = 16
NBUF = 4
E_PAD = NC * NS * C * K
N_ACC = 10240
ZR = N_ACC // NS

_MESH = plsc.VectorSubcoreMesh(core_axis_name="c", subcore_axis_name="s")


@functools.partial(
    pl.kernel,
    out_type=jax.ShapeDtypeStruct((NC, N_ACC, D), jnp.float32),
    mesh=_MESH,
    scratch_types=[
        pltpu.VMEM((2, IB, K), jnp.int32),
        pltpu.VMEM((2, IB, K), jnp.int32),
        [pltpu.VMEM((K, D), jnp.float32)] * NBUF,
        pltpu.VMEM_SHARED((N_ACC, D), jnp.float32),
        [pltpu.SemaphoreType.DMA] * NBUF,
        [pltpu.SemaphoreType.DMA] * NBUF,
        pltpu.SemaphoreType.DMA,
        pltpu.SemaphoreType.DMA,
    ],
)
def _sc_aggregate(s_hbm, src_hbm, dst_hbm, out_hbm,
                  src_v, dst_v, bufs, acc,
                  gsem, ssem, isem1, isem2):
    c = lax.axis_index("c")
    s = lax.axis_index("s")
    wid = c * NS + s
    buf0 = bufs[0]

    zero = jnp.zeros((16,), jnp.float32)

    def _zbuf(i, _):
        buf0[i // (D // 16), pl.ds((i % (D // 16)) * 16, 16)] = zero
        return 0
    lax.fori_loop(0, K * (D // 16), _zbuf, 0)

    for r in range(ZR // K):
        pltpu.sync_copy(buf0, acc.at[pl.ds(s * ZR + r * K, K)])
    assert ZR % K == 0
    plsc.subcore_barrier()

    def _round(p_sc, p_g, scat_base, gath_base):
        scats = []
        for b in range(NBUF):
            pltpu.make_async_copy(s_hbm.at[pl.ds(0, K)], bufs[b],
                                  gsem[b]).wait()
            scats.append(pltpu.async_copy(
                bufs[b], acc.at[dst_v.at[p_sc].at[scat_base + b]], ssem[b],
                add=True))
        for b in range(NBUF):
            scats[b].wait()
            pltpu.async_copy(s_hbm.at[src_v.at[p_g].at[gath_base + b]],
                             bufs[b], gsem[b])

    def _blockbody(p, bidx):
        base_next = wid * C + (bidx + 1) * IB
        r1 = pltpu.async_copy(src_hbm.at[pl.ds(base_next, IB)],
                              src_v.at[1 - p], isem1)
        r2 = pltpu.async_copy(dst_hbm.at[pl.ds(base_next, IB)],
                              dst_v.at[1 - p], isem2)

        def _q(q, _):
            _round(p, p, NBUF * q, NBUF * q + NBUF)
            return 0
        lax.fori_loop(0, IB // NBUF - 1, _q, 0)
        r1.wait()
        r2.wait()
        _round(p, 1 - p, IB - NBUF, 0)

    pltpu.sync_copy(src_hbm.at[pl.ds(wid * C, IB)], src_v.at[0])
    pltpu.sync_copy(dst_hbm.at[pl.ds(wid * C, IB)], dst_v.at[0])
    for b in range(NBUF):
        pltpu.async_copy(s_hbm.at[src_v.at[0].at[b]], bufs[b], gsem[b])

    def _pair(v, _):
        _blockbody(0, 2 * v)
        _blockbody(1, 2 * v + 1)
        return 0
    lax.fori_loop(0, C // IB // 2, _pair, 0)

    for b in range(NBUF):
        pltpu.make_async_copy(s_hbm.at[pl.ds(0, K)], bufs[b], gsem[b]).wait()

    plsc.subcore_barrier()

    pltpu.sync_copy(acc.at[pl.ds(s * ZR, ZR)], out_hbm.at[c].at[pl.ds(s * ZR, ZR)])


def _tc12_body(agg_ref, w1_ref, b1_ref, w2_ref, out_ref):
    a = agg_ref[0] + agg_ref[1]
    h = jnp.maximum(
        jnp.dot(a, w1_ref[...], preferred_element_type=jnp.float32,
                precision=lax.Precision.HIGHEST) + b1_ref[...], 0.0)
    out_ref[...] = jnp.dot(h, w2_ref[...], preferred_element_type=jnp.float32,
                           precision=lax.Precision.HIGHEST)


def _tc3_body(agg_ref, b2_ref, w3_ref, out_ref):
    h = jnp.maximum(agg_ref[0] + agg_ref[1] + b2_ref[...], 0.0)
    out_ref[...] = jnp.dot(h, w3_ref[...], preferred_element_type=jnp.float32,
                           precision=lax.Precision.HIGHEST)


def _tc_final_body(agg_ref, b3_ref, out_ref):
    out_ref[...] = agg_ref[0] + agg_ref[1] + b3_ref[...]


_R = 1000


def _tc12(aggx, w1, b1, w2):
    h1w = w1.shape[1]
    return pl.pallas_call(
        _tc12_body,
        grid=(N // _R,),
        in_specs=[
            pl.BlockSpec((NC, _R, D), lambda i: (0, i, 0)),
            pl.BlockSpec((D, h1w), lambda i: (0, 0)),
            pl.BlockSpec((1, h1w), lambda i: (0, 0)),
            pl.BlockSpec((h1w, D), lambda i: (0, 0)),
        ],
        out_specs=pl.BlockSpec((_R, D), lambda i: (i, 0)),
        out_shape=jax.ShapeDtypeStruct((N, D), jnp.float32),
    )(aggx, w1, b1.reshape(1, h1w), w2)


def _tc3(agg2, b2, w3):
    return pl.pallas_call(
        _tc3_body,
        grid=(N // _R,),
        in_specs=[
            pl.BlockSpec((NC, _R, D), lambda i: (0, i, 0)),
            pl.BlockSpec((1, D), lambda i: (0, 0)),
            pl.BlockSpec((D, D), lambda i: (0, 0)),
        ],
        out_specs=pl.BlockSpec((_R, D), lambda i: (i, 0)),
        out_shape=jax.ShapeDtypeStruct((N, D), jnp.float32),
    )(agg2, b2.reshape(1, D), w3)


def _tc_final(agg3, b3):
    return pl.pallas_call(
        _tc_final_body,
        grid=(N // _R,),
        in_specs=[
            pl.BlockSpec((NC, _R, D), lambda i: (0, i, 0)),
            pl.BlockSpec((1, D), lambda i: (0, 0)),
        ],
        out_specs=pl.BlockSpec((_R, D), lambda i: (i, 0)),
        out_shape=jax.ShapeDtypeStruct((N, D), jnp.float32),
    )(agg3, b3.reshape(1, D))


def kernel(x, adj, W1, b1, W2, b2, W3, b3):
    src = adj[0]
    dst = adj[1]
    nw = NC * NS
    per = E // nw
    pad_per = C * K - per
    src_p = jnp.pad(src.reshape(nw, per), ((0, 0), (0, pad_per))).reshape(-1, K)
    dummy = jnp.broadcast_to(
        (N + 16 + jnp.arange(nw, dtype=jnp.int32))[:, None], (nw, pad_per))
    dst_p = jnp.concatenate(
        [dst.reshape(nw, per), dummy], axis=1).reshape(-1, K)
    src_p = jnp.pad(src_p, ((0, IB), (0, 0)))
    dst_p = jnp.pad(dst_p, ((0, IB), (0, 0)))

    aggx = _sc_aggregate(x, src_p, dst_p)
    s2 = _tc12(aggx, W1, b1, W2)
    agg2 = _sc_aggregate(s2, src_p, dst_p)
    s3 = _tc3(agg2, b2, W3)
    agg3 = _sc_aggregate(s3, src_p, dst_p)
    return _tc_final(agg3, b3)


# speedup vs baseline: 1.0659x; 1.0024x over previous
import functools

import jax
import jax.numpy as jnp
from jax import lax
from jax.experimental import pallas as pl
from jax.experimental.pallas import tpu as pltpu
from jax.experimental.pallas import tpu_sc as plsc

N = 10000
E = 320000
D = 128

NC = 2
NS = 16
K = 64
C = 160
IB = 32
NBUF = 4
E_PAD = NC * NS * C * K
N_ACC = 10240
ZR = N_ACC // NS

_MESH = plsc.VectorSubcoreMesh(core_axis_name="c", subcore_axis_name="s")


@functools.partial(
    pl.kernel,
    out_type=jax.ShapeDtypeStruct((NC, N_ACC, D), jnp.float32),
    mesh=_MESH,
    scratch_types=[
        pltpu.VMEM((2, IB, K), jnp.int32),
        pltpu.VMEM((2, IB, K), jnp.int32),
        [pltpu.VMEM((K, D), jnp.float32)] * NBUF,
        pltpu.VMEM_SHARED((N_ACC, D), jnp.float32),
        [pltpu.SemaphoreType.DMA] * NBUF,
        [pltpu.SemaphoreType.DMA] * NBUF,
        pltpu.SemaphoreType.DMA,
        pltpu.SemaphoreType.DMA,
    ],
)
def _sc_aggregate(s_hbm, src_hbm, dst_hbm, out_hbm,
                  src_v, dst_v, bufs, acc,
                  gsem, ssem, isem1, isem2):
    c = lax.axis_index("c")
    s = lax.axis_index("s")
    wid = c * NS + s
    buf0 = bufs[0]

    zero = jnp.zeros((16,), jnp.float32)

    def _zbuf(i, _):
        buf0[i // (D // 16), pl.ds((i % (D // 16)) * 16, 16)] = zero
        return 0
    lax.fori_loop(0, K * (D // 16), _zbuf, 0)

    for r in range(ZR // K):
        pltpu.sync_copy(buf0, acc.at[pl.ds(s * ZR + r * K, K)])
    assert ZR % K == 0
    plsc.subcore_barrier()

    def _round(p_sc, p_g, scat_base, gath_base):
        scats = []
        for b in range(NBUF):
            pltpu.make_async_copy(s_hbm.at[pl.ds(0, K)], bufs[b],
                                  gsem[b]).wait()
            scats.append(pltpu.async_copy(
                bufs[b], acc.at[dst_v.at[p_sc].at[scat_base + b]], ssem[b],
                add=True))
        for b in range(NBUF):
            scats[b].wait()
            pltpu.async_copy(s_hbm.at[src_v.at[p_g].at[gath_base + b]],
                             bufs[b], gsem[b])

    def _blockbody(p, bidx):
        base_next = wid * C + (bidx + 1) * IB
        r1 = pltpu.async_copy(src_hbm.at[pl.ds(base_next, IB)],
                              src_v.at[1 - p], isem1)
        r2 = pltpu.async_copy(dst_hbm.at[pl.ds(base_next, IB)],
                              dst_v.at[1 - p], isem2)

        def _q(q, _):
            _round(p, p, NBUF * q, NBUF * q + NBUF)
            return 0
        lax.fori_loop(0, IB // NBUF - 1, _q, 0)
        r1.wait()
        r2.wait()
        _round(p, 1 - p, IB - NBUF, 0)

    pltpu.sync_copy(src_hbm.at[pl.ds(wid * C, IB)], src_v.at[0])
    pltpu.sync_copy(dst_hbm.at[pl.ds(wid * C, IB)], dst_v.at[0])
    for b in range(NBUF):
        pltpu.async_copy(s_hbm.at[src_v.at[0].at[b]], bufs[b], gsem[b])

    def _pair(v, _):
        _blockbody(0, 2 * v)
        _blockbody(1, 2 * v + 1)
        return 0
    lax.fori_loop(0, C // IB // 2, _pair, 0)
    if (C // IB) % 2:
        _blockbody(0, C // IB - 1)

    for b in range(NBUF):
        pltpu.make_async_copy(s_hbm.at[pl.ds(0, K)], bufs[b], gsem[b]).wait()

    plsc.subcore_barrier()

    pltpu.sync_copy(acc.at[pl.ds(s * ZR, ZR)], out_hbm.at[c].at[pl.ds(s * ZR, ZR)])


def _tc12_body(agg_ref, w1_ref, b1_ref, w2_ref, out_ref):
    a = agg_ref[0] + agg_ref[1]
    h = jnp.maximum(
        jnp.dot(a, w1_ref[...], preferred_element_type=jnp.float32,
                precision=lax.Precision.HIGHEST) + b1_ref[...], 0.0)
    out_ref[...] = jnp.dot(h, w2_ref[...], preferred_element_type=jnp.float32,
                           precision=lax.Precision.HIGHEST)


def _tc3_body(agg_ref, b2_ref, w3_ref, out_ref):
    h = jnp.maximum(agg_ref[0] + agg_ref[1] + b2_ref[...], 0.0)
    out_ref[...] = jnp.dot(h, w3_ref[...], preferred_element_type=jnp.float32,
                           precision=lax.Precision.HIGHEST)


def _tc_final_body(agg_ref, b3_ref, out_ref):
    out_ref[...] = agg_ref[0] + agg_ref[1] + b3_ref[...]


_R = 1000


def _tc12(aggx, w1, b1, w2):
    h1w = w1.shape[1]
    return pl.pallas_call(
        _tc12_body,
        grid=(N // _R,),
        in_specs=[
            pl.BlockSpec((NC, _R, D), lambda i: (0, i, 0)),
            pl.BlockSpec((D, h1w), lambda i: (0, 0)),
            pl.BlockSpec((1, h1w), lambda i: (0, 0)),
            pl.BlockSpec((h1w, D), lambda i: (0, 0)),
        ],
        out_specs=pl.BlockSpec((_R, D), lambda i: (i, 0)),
        out_shape=jax.ShapeDtypeStruct((N, D), jnp.float32),
    )(aggx, w1, b1.reshape(1, h1w), w2)


def _tc3(agg2, b2, w3):
    return pl.pallas_call(
        _tc3_body,
        grid=(N // _R,),
        in_specs=[
            pl.BlockSpec((NC, _R, D), lambda i: (0, i, 0)),
            pl.BlockSpec((1, D), lambda i: (0, 0)),
            pl.BlockSpec((D, D), lambda i: (0, 0)),
        ],
        out_specs=pl.BlockSpec((_R, D), lambda i: (i, 0)),
        out_shape=jax.ShapeDtypeStruct((N, D), jnp.float32),
    )(agg2, b2.reshape(1, D), w3)


def _tc_final(agg3, b3):
    return pl.pallas_call(
        _tc_final_body,
        grid=(N // _R,),
        in_specs=[
            pl.BlockSpec((NC, _R, D), lambda i: (0, i, 0)),
            pl.BlockSpec((1, D), lambda i: (0, 0)),
        ],
        out_specs=pl.BlockSpec((_R, D), lambda i: (i, 0)),
        out_shape=jax.ShapeDtypeStruct((N, D), jnp.float32),
    )(agg3, b3.reshape(1, D))


def kernel(x, adj, W1, b1, W2, b2, W3, b3):
    src = adj[0]
    dst = adj[1]
    nw = NC * NS
    per = E // nw
    pad_per = C * K - per
    src_p = jnp.pad(src.reshape(nw, per), ((0, 0), (0, pad_per))).reshape(-1, K)
    dummy = jnp.broadcast_to(
        (N + 16 + jnp.arange(nw, dtype=jnp.int32))[:, None], (nw, pad_per))
    dst_p = jnp.concatenate(
        [dst.reshape(nw, per), dummy], axis=1).reshape(-1, K)
    src_p = jnp.pad(src_p, ((0, IB), (0, 0)))
    dst_p = jnp.pad(dst_p, ((0, IB), (0, 0)))

    aggx = _sc_aggregate(x, src_p, dst_p)
    s2 = _tc12(aggx, W1, b1, W2)
    agg2 = _sc_aggregate(s2, src_p, dst_p)
    s3 = _tc3(agg2, b2, W3)
    agg3 = _sc_aggregate(s3, src_p, dst_p)
    return _tc_final(agg3, b3)
